# baseline (speedup 1.0000x reference)
import jax
import jax.numpy as jnp
from jax.experimental import pallas as pl
from jax.experimental.pallas import tpu as pltpu

_TOPK = 96
_G = 2


def _krause_kernel(x_ref, wq_hbm, wk_hbm, wv_hbm, wo_hbm,
                   bq_ref, bk_ref, bv_ref, bo_ref, ls_ref, out_ref,
                   gates_ref, wq_scr, wk_scr, wv_scr, wo_scr,
                   sem_q, sem_k, sem_v, sem_o):
    gd = wv_scr.shape[0]
    gd2 = gd // 2
    dh = gd // _G
    nq = gates_ref.shape[1] // gd
    dn = (((1,), (1,)), ((), ()))
    x = x_ref[...]

    def copy_pair(s, slot):
        cps = []
        for (hbm, scr, sem) in ((wq_hbm, wq_scr, sem_q), (wk_hbm, wk_scr, sem_k)):
            cps.append(pltpu.make_async_copy(
                hbm.at[pl.ds(s * gd, gd2), :],
                scr.at[slot, pl.ds(0, gd2), :], sem.at[slot, 0]))
            cps.append(pltpu.make_async_copy(
                hbm.at[pl.ds(s * gd + gd2, gd2), :],
                scr.at[slot, pl.ds(gd2, gd2), :], sem.at[slot, 1]))
        return cps

    opened = jnp.zeros((), jnp.int32)
    for cp in copy_pair(0, 0):
        cp.start()
    for s in range(nq):
        if s + 1 < nq:
            for cp in copy_pair(s + 1, (s + 1) % 2):
                cp.start()
        slot = s % 2
        for cp in copy_pair(s, slot):
            cp.wait()
        q = jax.lax.dot_general(x, wq_scr[slot], dn,
                                preferred_element_type=jnp.float32) + bq_ref[0, :, s * gd:(s + 1) * gd]
        k = jax.lax.dot_general(x, wk_scr[slot], dn,
                                preferred_element_type=jnp.float32) + bk_ref[0, :, s * gd:(s + 1) * gd]

        for hh in range(_G):
            qh = q[:, hh * dh:(hh + 1) * dh]
            kh = k[:, hh * dh:(hh + 1) * dh]
            d_real = jnp.sum((qh - kh) ** 2, axis=1, keepdims=True)
            d_zero = jnp.sum(qh * qh, axis=1, keepdims=True)
            ls = ls_ref[s * _G + hh, 0, 0]
            inv_two_sigma_sq = 0.5 * jnp.exp(-2.0 * ls)
            z = (d_real - d_zero) * inv_two_sigma_sq
            gate = jnp.where(d_real < d_zero,
                             1.0 / (1.0 + (_TOPK - 1) * jnp.exp(z)),
                             0.0)
            gates_ref[:, s * gd + hh * dh:s * gd + (hh + 1) * dh] = (
                jnp.broadcast_to(gate, (gate.shape[0], dh)))
            n_open = jnp.sum(jnp.where(d_real < d_zero, 1.0, 0.0))
            opened = opened | (n_open > 0).astype(jnp.int32)

    out_ref[...] = jnp.broadcast_to(bo_ref[...], out_ref.shape)

    @pl.when(opened > 0)
    def _open_path():
        def slab(s, _):
            cp_v = pltpu.make_async_copy(
                wv_hbm.at[pl.ds(s * gd, gd), :], wv_scr, sem_v)
            cp_o = pltpu.make_async_copy(
                wo_hbm.at[:, pl.ds(s * gd, gd)], wo_scr, sem_o)
            cp_v.start()
            cp_o.start()
            cp_v.wait()
            cp_o.wait()
            v = (jax.lax.dot_general(x, wv_scr[...], dn,
                                     preferred_element_type=jnp.float32)
                 + bv_ref[:, pl.ds(s * gd, gd)])
            y = v * gates_ref[:, pl.ds(s * gd, gd)]
            out_ref[...] += jax.lax.dot_general(
                y, wo_scr[...], dn, preferred_element_type=jnp.float32)
            return 0

        jax.lax.fori_loop(0, nq, slab, 0)


def kernel(x, Wq, bq, Wk, bk, Wv, bv, Wo, bo, log_sigma, current_pos):
    del current_pos
    B, T, E = x.shape
    H = log_sigma.shape[0]
    DH = E // H
    GD = _G * DH

    xf = x.reshape(B, E)
    bq2 = bq.reshape(1, 1, E)
    bk2 = bk.reshape(1, 1, E)
    bv2 = bv.reshape(1, E)
    bo2 = bo.reshape(1, E)
    ls2 = log_sigma.reshape(H, 1, 1)

    out = pl.pallas_call(
        _krause_kernel,
        grid=(1,),
        in_specs=[
            pl.BlockSpec((B, E), lambda i: (0, 0)),
            pl.BlockSpec(memory_space=pltpu.MemorySpace.HBM),
            pl.BlockSpec(memory_space=pltpu.MemorySpace.HBM),
            pl.BlockSpec(memory_space=pltpu.MemorySpace.HBM),
            pl.BlockSpec(memory_space=pltpu.MemorySpace.HBM),
            pl.BlockSpec((1, 1, E), lambda i: (0, 0, 0)),
            pl.BlockSpec((1, 1, E), lambda i: (0, 0, 0)),
            pl.BlockSpec((1, E), lambda i: (0, 0)),
            pl.BlockSpec((1, E), lambda i: (0, 0)),
            pl.BlockSpec((H, 1, 1), lambda i: (0, 0, 0)),
        ],
        out_specs=pl.BlockSpec((B, E), lambda i: (0, 0)),
        out_shape=jax.ShapeDtypeStruct((B, E), jnp.float32),
        scratch_shapes=[
            pltpu.VMEM((B, E), jnp.float32),
            pltpu.VMEM((2, GD, E), jnp.float32),
            pltpu.VMEM((2, GD, E), jnp.float32),
            pltpu.VMEM((GD, E), jnp.float32),
            pltpu.VMEM((E, GD), jnp.float32),
            pltpu.SemaphoreType.DMA((2, 2)),
            pltpu.SemaphoreType.DMA((2, 2)),
            pltpu.SemaphoreType.DMA,
            pltpu.SemaphoreType.DMA,
        ],
        compiler_params=pltpu.CompilerParams(
            dimension_semantics=("arbitrary",)),
    )(xf, Wq, Wk, Wv, Wo, bq2, bk2, bv2, bo2, ls2)

    return out.reshape(B, 1, E)


# speedup vs baseline: 1.0338x; 1.0338x over previous
import jax
import jax.numpy as jnp
from jax.experimental import pallas as pl
from jax.experimental.pallas import tpu as pltpu

_TOPK = 96
_G = 4


def _krause_kernel(x_ref, wq_hbm, wk_hbm, wv_hbm, wo_hbm,
                   bq_ref, bk_ref, bv_ref, bo_ref, ls_ref, out_ref,
                   gates_ref, wq_scr, wk_scr, wv_scr, wo_scr,
                   sem_q, sem_k, sem_v, sem_o):
    gd = wv_scr.shape[0]
    gd2 = gd // 2
    dh = gd // _G
    nq = gates_ref.shape[1] // gd
    dn = (((1,), (1,)), ((), ()))
    x = x_ref[...]

    def copy_pair(s, slot):
        cps = []
        for (hbm, scr, sem) in ((wq_hbm, wq_scr, sem_q), (wk_hbm, wk_scr, sem_k)):
            cps.append(pltpu.make_async_copy(
                hbm.at[pl.ds(s * gd, gd2), :],
                scr.at[slot, pl.ds(0, gd2), :], sem.at[slot, 0]))
            cps.append(pltpu.make_async_copy(
                hbm.at[pl.ds(s * gd + gd2, gd2), :],
                scr.at[slot, pl.ds(gd2, gd2), :], sem.at[slot, 1]))
        return cps

    opened = jnp.zeros((), jnp.int32)
    for s in range(nq):
        for cp in copy_pair(s, s):
            cp.start()
    for s in range(nq):
        slot = s
        for cp in copy_pair(s, slot):
            cp.wait()
        q = jax.lax.dot_general(x, wq_scr[slot], dn,
                                preferred_element_type=jnp.float32) + bq_ref[0, :, s * gd:(s + 1) * gd]
        k = jax.lax.dot_general(x, wk_scr[slot], dn,
                                preferred_element_type=jnp.float32) + bk_ref[0, :, s * gd:(s + 1) * gd]

        for hh in range(_G):
            qh = q[:, hh * dh:(hh + 1) * dh]
            kh = k[:, hh * dh:(hh + 1) * dh]
            d_real = jnp.sum((qh - kh) ** 2, axis=1, keepdims=True)
            d_zero = jnp.sum(qh * qh, axis=1, keepdims=True)
            ls = ls_ref[s * _G + hh, 0, 0]
            inv_two_sigma_sq = 0.5 * jnp.exp(-2.0 * ls)
            z = (d_real - d_zero) * inv_two_sigma_sq
            gate = jnp.where(d_real < d_zero,
                             1.0 / (1.0 + (_TOPK - 1) * jnp.exp(z)),
                             0.0)
            gates_ref[:, s * gd + hh * dh:s * gd + (hh + 1) * dh] = (
                jnp.broadcast_to(gate, (gate.shape[0], dh)))
            n_open = jnp.sum(jnp.where(d_real < d_zero, 1.0, 0.0))
            opened = opened | (n_open > 0).astype(jnp.int32)

    out_ref[...] = jnp.broadcast_to(bo_ref[...], out_ref.shape)

    @pl.when(opened > 0)
    def _open_path():
        def slab(s, _):
            cp_v = pltpu.make_async_copy(
                wv_hbm.at[pl.ds(s * gd, gd), :], wv_scr, sem_v)
            cp_o = pltpu.make_async_copy(
                wo_hbm.at[:, pl.ds(s * gd, gd)], wo_scr, sem_o)
            cp_v.start()
            cp_o.start()
            cp_v.wait()
            cp_o.wait()
            v = (jax.lax.dot_general(x, wv_scr[...], dn,
                                     preferred_element_type=jnp.float32)
                 + bv_ref[:, pl.ds(s * gd, gd)])
            y = v * gates_ref[:, pl.ds(s * gd, gd)]
            out_ref[...] += jax.lax.dot_general(
                y, wo_scr[...], dn, preferred_element_type=jnp.float32)
            return 0

        jax.lax.fori_loop(0, nq, slab, 0)


def kernel(x, Wq, bq, Wk, bk, Wv, bv, Wo, bo, log_sigma, current_pos):
    del current_pos
    B, T, E = x.shape
    H = log_sigma.shape[0]
    DH = E // H
    GD = _G * DH

    xf = x.reshape(B, E)
    bq2 = bq.reshape(1, 1, E)
    bk2 = bk.reshape(1, 1, E)
    bv2 = bv.reshape(1, E)
    bo2 = bo.reshape(1, E)
    ls2 = log_sigma.reshape(H, 1, 1)

    out = pl.pallas_call(
        _krause_kernel,
        grid=(1,),
        in_specs=[
            pl.BlockSpec((B, E), lambda i: (0, 0)),
            pl.BlockSpec(memory_space=pltpu.MemorySpace.HBM),
            pl.BlockSpec(memory_space=pltpu.MemorySpace.HBM),
            pl.BlockSpec(memory_space=pltpu.MemorySpace.HBM),
            pl.BlockSpec(memory_space=pltpu.MemorySpace.HBM),
            pl.BlockSpec((1, 1, E), lambda i: (0, 0, 0)),
            pl.BlockSpec((1, 1, E), lambda i: (0, 0, 0)),
            pl.BlockSpec((1, E), lambda i: (0, 0)),
            pl.BlockSpec((1, E), lambda i: (0, 0)),
            pl.BlockSpec((H, 1, 1), lambda i: (0, 0, 0)),
        ],
        out_specs=pl.BlockSpec((B, E), lambda i: (0, 0)),
        out_shape=jax.ShapeDtypeStruct((B, E), jnp.float32),
        scratch_shapes=[
            pltpu.VMEM((B, E), jnp.float32),
            pltpu.VMEM((H // _G, GD, E), jnp.float32),
            pltpu.VMEM((H // _G, GD, E), jnp.float32),
            pltpu.VMEM((GD, E), jnp.float32),
            pltpu.VMEM((E, GD), jnp.float32),
            pltpu.SemaphoreType.DMA((H // _G, 2)),
            pltpu.SemaphoreType.DMA((H // _G, 2)),
            pltpu.SemaphoreType.DMA,
            pltpu.SemaphoreType.DMA,
        ],
        compiler_params=pltpu.CompilerParams(
            dimension_semantics=("arbitrary",)),
    )(xf, Wq, Wk, Wv, Wo, bq2, bk2, bv2, bo2, ls2)

    return out.reshape(B, 1, E)
